# baseline (speedup 1.0000x reference)
import functools

import jax
import jax.numpy as jnp
from jax import lax
from jax.experimental import pallas as pl
from jax.experimental.pallas import tpu as pltpu
from jax.experimental.pallas import tpu_sc as plsc

_NC = 2
_NS = 16
_NW = _NC * _NS

_B = 8
_H = 28
_HO = 14
_N = _H * _H
_M = _HO * _HO
_C = 16
_ROW = _N * _C
_OROW = _M * _C
_STRIP = (_B * _M) // _NW


def _sc_body(mu_hbm, sig_hbm, muo_hbm, sigo_hbm, mub, wtab, murow, quad,
             outrow, sem):
    cid = lax.axis_index("c")
    sid = lax.axis_index("s")
    wid = sid * _NC + cid
    b = wid // 4
    m1lo = (wid % 4) * _STRIP
    iota = lax.iota(jnp.int32, 16)

    pltpu.sync_copy(mu_hbm.at[pl.ds(b * (_H * _H * _C), _H * _H * _C)], mub)

    def build(m2, _):
        oy = m2 // _HO
        ox = m2 % _HO
        base = oy * (2 * _H * _C) + ox * (2 * _C)
        v00 = mub[pl.ds(base, 16)]
        v01 = mub[pl.ds(base + _C, 16)]
        v10 = mub[pl.ds(base + _H * _C, 16)]
        v11 = mub[pl.ds(base + _H * _C + _C, 16)]
        m = jnp.maximum(jnp.maximum(v00, v01), jnp.maximum(v10, v11))
        n00 = oy * (2 * _H) + 2 * ox
        n00v = jnp.full((16,), n00 * _C, jnp.int32)
        c16 = jnp.full((16,), _C, jnp.int32)
        hc = jnp.full((16,), _H * _C, jnp.int32)
        i2c = jnp.where(
            v00 == m, n00v,
            jnp.where(v01 == m, n00v + c16,
                      jnp.where(v10 == m, n00v + hc, n00v + hc + c16)))
        wtab[pl.ds(m2 * _C, 16)] = i2c + iota
        rel = m2 - m1lo

        @pl.when((rel >= 0) & (rel < _STRIP))
        def _():
            murow[pl.ds(rel * _C, 16)] = m

        return 0

    lax.fori_loop(0, _M, build, 0)
    pltpu.sync_copy(murow,
                    muo_hbm.at[pl.ds(b * _OROW + m1lo * _C, _STRIP * _C)])

    def pair(j, _):
        m1 = m1lo + j
        oy = m1 // _HO
        ox = m1 % _HO
        nb = oy * (2 * _H) + 2 * ox
        rb = (b * _N + nb) * _ROW
        cp0 = pltpu.async_copy(sig_hbm.at[pl.ds(rb, _ROW)],
                               quad.at[pl.ds(0, _ROW)], sem)
        cp1 = pltpu.async_copy(sig_hbm.at[pl.ds(rb + _ROW, _ROW)],
                               quad.at[pl.ds(_ROW, _ROW)], sem)
        cp2 = pltpu.async_copy(sig_hbm.at[pl.ds(rb + _H * _ROW, _ROW)],
                               quad.at[pl.ds(2 * _ROW, _ROW)], sem)
        cp3 = pltpu.async_copy(sig_hbm.at[pl.ds(rb + (_H + 1) * _ROW, _ROW)],
                               quad.at[pl.ds(3 * _ROW, _ROW)], sem)
        cp0.wait()
        cp1.wait()
        cp2.wait()
        cp3.wait()
        r16 = wtab[pl.ds(m1 * _C, 16)] - iota - jnp.full(
            (16,), nb * _C, jnp.int32)
        dy = r16 // jnp.full((16,), _H * _C, jnp.int32)
        dx = lax.shift_right_logical(
            r16, jnp.full((16,), 4, jnp.int32)) & jnp.full((16,), 1, jnp.int32)
        qoff = (dy + dy + dx) * jnp.full((16,), _ROW, jnp.int32)

        def inner(m2, _):
            idx = qoff + wtab[pl.ds(m2 * _C, 16)]
            outrow[pl.ds(m2 * _C, 16)] = plsc.load_gather(quad, [idx])
            return 0

        lax.fori_loop(0, _M, inner, 0)
        pltpu.sync_copy(outrow,
                        sigo_hbm.at[pl.ds((b * _M + m1) * _OROW, _OROW)])
        return 0

    lax.fori_loop(0, _STRIP, pair, 0)


@jax.jit
def _vdp_pool(mu_r, sig_r):
    mesh = plsc.VectorSubcoreMesh(core_axis_name="c", subcore_axis_name="s",
                                  num_cores=_NC, num_subcores=_NS)
    return pl.kernel(
        _sc_body,
        out_type=(
            jax.ShapeDtypeStruct((_B * _M * _C,), jnp.float32),
            jax.ShapeDtypeStruct((_B * _M * _OROW,), jnp.float32),
        ),
        mesh=mesh,
        compiler_params=pltpu.CompilerParams(needs_layout_passes=False),
        scratch_types=[
            pltpu.VMEM((_H * _H * _C,), jnp.float32),
            pltpu.VMEM((_M * _C,), jnp.int32),
            pltpu.VMEM((_STRIP * _C,), jnp.float32),
            pltpu.VMEM((4 * _ROW,), jnp.float32),
            pltpu.VMEM((_OROW,), jnp.float32),
            pltpu.SemaphoreType.DMA,
        ],
    )(mu_r, sig_r)


def kernel(mu_in, Sigma_in):
    mu_r = mu_in.reshape(_B * _H * _H * _C)
    sig_r = Sigma_in.reshape(_B * _N * _ROW)
    muo, sigo = _vdp_pool(mu_r, sig_r)
    mu_out = muo.reshape(_B, _HO, _HO, _C)
    Sigma_out = sigo.reshape(_B, _M, _M, _C)
    return mu_out, Sigma_out


# speedup vs baseline: 6.8786x; 6.8786x over previous
import jax
import jax.numpy as jnp
from jax import lax
from jax.experimental import pallas as pl
from jax.experimental.pallas import tpu as pltpu
from jax.experimental.pallas import tpu_sc as plsc

_NC = 2
_NS = 16
_NW = _NC * _NS

_B = 8
_H = 28
_HO = 14
_N = _H * _H
_M = _HO * _HO
_C = 16
_ROW = _N * _C
_OROW = _M * _C
_STRIP = (_B * _M) // _NW


def _sc_body(mu_hbm, sig_hbm, muo_hbm, sigo_hbm, mub, wtab, murow, quad,
             outrow, sem):
    cid = lax.axis_index("c")
    sid = lax.axis_index("s")
    wid = sid * _NC + cid
    b = wid // 4
    m1lo = (wid % 4) * _STRIP
    iota = lax.iota(jnp.int32, 16)

    pltpu.sync_copy(mu_hbm.at[pl.ds(b * (_H * _H * _C), _H * _H * _C)], mub)

    def build(m2, _):
        oy = m2 // _HO
        ox = m2 % _HO
        base = oy * (2 * _H * _C) + ox * (2 * _C)
        v00 = mub[pl.ds(base, 16)]
        v01 = mub[pl.ds(base + _C, 16)]
        v10 = mub[pl.ds(base + _H * _C, 16)]
        v11 = mub[pl.ds(base + _H * _C + _C, 16)]
        m = jnp.maximum(jnp.maximum(v00, v01), jnp.maximum(v10, v11))
        n00 = oy * (2 * _H) + 2 * ox
        n00v = jnp.full((16,), n00, jnp.int32)
        one = jnp.full((16,), 1, jnp.int32)
        hv = jnp.full((16,), _H, jnp.int32)
        i2 = jnp.where(
            v00 == m, n00v,
            jnp.where(v01 == m, n00v + one,
                      jnp.where(v10 == m, n00v + hv, n00v + hv + one)))
        wtab[pl.ds(m2 * _C, 16)] = i2
        rel = m2 - m1lo

        @pl.when((rel >= 0) & (rel < _STRIP))
        def _():
            murow[pl.ds(rel * _C, 16)] = m

        return 0

    lax.fori_loop(0, _M, build, 0)
    pltpu.sync_copy(murow,
                    muo_hbm.at[pl.ds(b * _OROW + m1lo * _C, _STRIP * _C)])

    def pair(j, _):
        m1 = m1lo + j
        oy = m1 // _HO
        ox = m1 % _HO
        nb = oy * (2 * _H) + 2 * ox
        cp0 = pltpu.async_copy(sig_hbm.at[b, nb], quad.at[0], sem)
        cp1 = pltpu.async_copy(sig_hbm.at[b, nb + 1], quad.at[1], sem)
        cp2 = pltpu.async_copy(sig_hbm.at[b, nb + _H], quad.at[2], sem)
        cp3 = pltpu.async_copy(sig_hbm.at[b, nb + _H + 1], quad.at[3], sem)
        cp0.wait()
        cp1.wait()
        cp2.wait()
        cp3.wait()
        r = wtab[pl.ds(m1 * _C, 16)] - jnp.full((16,), nb, jnp.int32)
        dy = r // jnp.full((16,), _H, jnp.int32)
        dx = r & jnp.full((16,), 1, jnp.int32)
        qsel = dy + dy + dx

        def inner(m2, _):
            i2v = wtab[pl.ds(m2 * _C, 16)]
            val = plsc.load_gather(quad, [qsel, iota, i2v])
            plsc.store_scatter(outrow, [iota, jnp.full((16,), m2, jnp.int32)],
                               val)
            return 0

        lax.fori_loop(0, _M, inner, 0)
        pltpu.sync_copy(outrow, sigo_hbm.at[b, m1])
        return 0

    lax.fori_loop(0, _STRIP, pair, 0)


@jax.jit
def _vdp_pool(mu_r, sig_t):
    mesh = plsc.VectorSubcoreMesh(core_axis_name="c", subcore_axis_name="s",
                                  num_cores=_NC, num_subcores=_NS)
    return pl.kernel(
        _sc_body,
        out_type=(
            jax.ShapeDtypeStruct((_B * _M * _C,), jnp.float32),
            jax.ShapeDtypeStruct((_B, _M, _C, _M), jnp.float32),
        ),
        mesh=mesh,
        compiler_params=pltpu.CompilerParams(needs_layout_passes=False),
        scratch_types=[
            pltpu.VMEM((_H * _H * _C,), jnp.float32),
            pltpu.VMEM((_M * _C,), jnp.int32),
            pltpu.VMEM((_STRIP * _C,), jnp.float32),
            pltpu.VMEM((4, _C, _N), jnp.float32),
            pltpu.VMEM((_C, _M), jnp.float32),
            pltpu.SemaphoreType.DMA,
        ],
    )(mu_r, sig_t)


def kernel(mu_in, Sigma_in):
    mu_r = mu_in.reshape(_B * _H * _H * _C)
    sig_t = jnp.transpose(Sigma_in, (0, 1, 3, 2))
    muo, sigo = _vdp_pool(mu_r, sig_t)
    mu_out = muo.reshape(_B, _HO, _HO, _C)
    Sigma_out = jnp.transpose(sigo, (0, 1, 3, 2))
    return mu_out, Sigma_out


# speedup vs baseline: 11.0454x; 1.6058x over previous
import jax
import jax.numpy as jnp
from jax import lax
from jax.experimental import pallas as pl
from jax.experimental.pallas import tpu as pltpu
from jax.experimental.pallas import tpu_sc as plsc

_NC = 2
_NS = 16
_NW = _NC * _NS

_B = 8
_H = 28
_HO = 14
_N = _H * _H
_M = _HO * _HO
_C = 16
_OROW = _M * _C
_STRIP = (_B * _M) // _NW


def _sc_body(mu_hbm, sig_hbm, muo_hbm, sigo_hbm, mub, wtab, murow, quad,
             outrow, semA, semB, semOA, semOB):
    cid = lax.axis_index("c")
    sid = lax.axis_index("s")
    wid = sid * _NC + cid
    b = wid // 4
    m1lo = (wid % 4) * _STRIP
    iota = lax.iota(jnp.int32, 16)

    def build_oy(oy, _):
        pltpu.sync_copy(
            mu_hbm.at[pl.ds(b * (_H * _H * _C) + oy * (2 * _H * _C),
                            2 * _H * _C)], mub)

        def build_ox(ox, _):
            m2 = oy * _HO + ox
            base = ox * (2 * _C)
            v00 = mub[pl.ds(base, 16)]
            v01 = mub[pl.ds(base + _C, 16)]
            v10 = mub[pl.ds(base + _H * _C, 16)]
            v11 = mub[pl.ds(base + _H * _C + _C, 16)]
            m = jnp.maximum(jnp.maximum(v00, v01), jnp.maximum(v10, v11))
            n00 = oy * (2 * _H) + 2 * ox
            n00v = jnp.full((16,), n00, jnp.int32)
            one = jnp.full((16,), 1, jnp.int32)
            hv = jnp.full((16,), _H, jnp.int32)
            i2 = jnp.where(
                v00 == m, n00v,
                jnp.where(v01 == m, n00v + one,
                          jnp.where(v10 == m, n00v + hv, n00v + hv + one)))
            wtab[pl.ds(m2 * _C, 16)] = i2
            rel = m2 - m1lo

            @pl.when((rel >= 0) & (rel < _STRIP))
            def _():
                murow[pl.ds(rel * _C, 16)] = m

            return 0

        lax.fori_loop(0, _HO, build_ox, 0)
        return 0

    lax.fori_loop(0, _HO, build_oy, 0)
    pltpu.sync_copy(murow,
                    muo_hbm.at[pl.ds(b * _OROW + m1lo * _C, _STRIP * _C)])

    def _nb(j):
        m1 = m1lo + j
        return (m1 // _HO) * (2 * _H) + 2 * (m1 % _HO)

    def _fire(j, sem):
        nb = _nb(j)
        h = j % 2
        pltpu.async_copy(sig_hbm.at[b, pl.ds(nb, 2)],
                         quad.at[h, pl.ds(0, 2)], sem)
        pltpu.async_copy(sig_hbm.at[b, pl.ds(nb + _H, 2)],
                         quad.at[h, pl.ds(2, 2)], sem)

    def _drain(j, sem):
        h = j % 2
        for k in range(2):
            pltpu.make_async_copy(sig_hbm.at[b, pl.ds(0, 2)],
                                  quad.at[h, pl.ds(2 * k, 2)], sem).wait()

    _fire(0, semA)

    def pair(j, _):
        even = (j % 2) == 0
        h = j % 2

        @pl.when(j + 1 < _STRIP)
        def _():
            @pl.when(even)
            def _():
                _fire(j + 1, semB)

            @pl.when(jnp.logical_not(even))
            def _():
                _fire(j + 1, semA)

        @pl.when(even)
        def _():
            _drain(j, semA)

        @pl.when(jnp.logical_not(even))
        def _():
            _drain(j, semB)

        m1 = m1lo + j
        nb = _nb(j)
        r = wtab[pl.ds(m1 * _C, 16)] - jnp.full((16,), nb, jnp.int32)
        dy = r // jnp.full((16,), _H, jnp.int32)
        dx = r & jnp.full((16,), 1, jnp.int32)
        qsel = dy + dy + dx
        hv16 = jnp.full((16,), h, jnp.int32)

        @pl.when((j >= 2) & even)
        def _():
            pltpu.make_async_copy(outrow.at[0], sigo_hbm.at[b, 0],
                                  semOA).wait()

        @pl.when((j >= 2) & jnp.logical_not(even))
        def _():
            pltpu.make_async_copy(outrow.at[1], sigo_hbm.at[b, 0],
                                  semOB).wait()

        def inner(m2, _):
            i2v = wtab[pl.ds(m2 * _C, 16)]
            val = plsc.load_gather(quad, [hv16, qsel, iota, i2v])
            plsc.store_scatter(
                outrow, [hv16, iota, jnp.full((16,), m2, jnp.int32)], val)
            return 0

        lax.fori_loop(0, _M, inner, 0)

        @pl.when(even)
        def _():
            pltpu.async_copy(outrow.at[0], sigo_hbm.at[b, m1], semOA)

        @pl.when(jnp.logical_not(even))
        def _():
            pltpu.async_copy(outrow.at[1], sigo_hbm.at[b, m1], semOB)

        return 0

    lax.fori_loop(0, _STRIP, pair, 0)
    pltpu.make_async_copy(outrow.at[0], sigo_hbm.at[b, 0], semOA).wait()
    pltpu.make_async_copy(outrow.at[1], sigo_hbm.at[b, 0], semOB).wait()


@jax.jit
def _vdp_pool(mu_r, sig_t):
    mesh = plsc.VectorSubcoreMesh(core_axis_name="c", subcore_axis_name="s",
                                  num_cores=_NC, num_subcores=_NS)
    return pl.kernel(
        _sc_body,
        out_type=(
            jax.ShapeDtypeStruct((_B * _M * _C,), jnp.float32),
            jax.ShapeDtypeStruct((_B, _M, _C, _M), jnp.float32),
        ),
        mesh=mesh,
        compiler_params=pltpu.CompilerParams(needs_layout_passes=False),
        scratch_types=[
            pltpu.VMEM((2 * _H * _C,), jnp.float32),
            pltpu.VMEM((_M * _C,), jnp.int32),
            pltpu.VMEM((_STRIP * _C,), jnp.float32),
            pltpu.VMEM((2, 4, _C, _N), jnp.float32),
            pltpu.VMEM((2, _C, _M), jnp.float32),
            pltpu.SemaphoreType.DMA,
            pltpu.SemaphoreType.DMA,
            pltpu.SemaphoreType.DMA,
            pltpu.SemaphoreType.DMA,
        ],
    )(mu_r, sig_t)


def kernel(mu_in, Sigma_in):
    mu_r = mu_in.reshape(_B * _H * _H * _C)
    sig_t = jnp.transpose(Sigma_in, (0, 1, 3, 2))
    muo, sigo = _vdp_pool(mu_r, sig_t)
    mu_out = muo.reshape(_B, _HO, _HO, _C)
    Sigma_out = jnp.transpose(sigo, (0, 1, 3, 2))
    return mu_out, Sigma_out


# speedup vs baseline: 12.6389x; 1.1443x over previous
import jax
import jax.numpy as jnp
from jax import lax
from jax.experimental import pallas as pl
from jax.experimental.pallas import tpu as pltpu
from jax.experimental.pallas import tpu_sc as plsc

_NC = 2
_NS = 16
_NW = _NC * _NS

_B = 8
_H = 28
_HO = 14
_N = _H * _H
_M = _HO * _HO
_C = 16
_OROW = _M * _C
_STRIP = (_B * _M) // _NW


_NCH = (_M + 15) // 16


def _sc_body(mu_hbm, sig_hbm, muo_hbm, sigo_hbm, mub, wtab, murow, quad,
             outrow, semA, semB, semOA, semOB):
    cid = lax.axis_index("c")
    sid = lax.axis_index("s")
    wid = sid * _NC + cid
    b = wid // 4
    m1lo = (wid % 4) * _STRIP
    iota = lax.iota(jnp.int32, 16)

    def _nb(j):
        m1 = m1lo + j
        return (m1 // _HO) * (2 * _H) + 2 * (m1 % _HO)

    nb0 = _nb(0)
    pltpu.async_copy(sig_hbm.at[b, pl.ds(nb0, 2)],
                     quad.at[0, pl.ds(0, 2)], semA)
    pltpu.async_copy(sig_hbm.at[b, pl.ds(nb0 + _H, 2)],
                     quad.at[0, pl.ds(2, 2)], semA)

    def _mu_src(oy):
        return mu_hbm.at[pl.ds(b * (_H * _H * _C) + oy * (2 * _H * _C),
                               2 * _H * _C)]

    pltpu.async_copy(_mu_src(0), mub.at[0], semB)

    def build_oy(oy, _):
        @pl.when(oy + 1 < _HO)
        def _():
            pltpu.async_copy(_mu_src(oy + 1), mub.at[(oy + 1) % 2], semB)

        g = oy % 2
        pltpu.make_async_copy(_mu_src(0), mub.at[g], semB).wait()

        def build_ox(ox, _):
            m2 = oy * _HO + ox
            base = ox * (2 * _C)
            v00 = mub[g, pl.ds(base, 16)]
            v01 = mub[g, pl.ds(base + _C, 16)]
            v10 = mub[g, pl.ds(base + _H * _C, 16)]
            v11 = mub[g, pl.ds(base + _H * _C + _C, 16)]
            m = jnp.maximum(jnp.maximum(v00, v01), jnp.maximum(v10, v11))
            n00 = oy * (2 * _H) + 2 * ox
            n00v = jnp.full((16,), n00, jnp.int32)
            one = jnp.full((16,), 1, jnp.int32)
            hv = jnp.full((16,), _H, jnp.int32)
            i2 = jnp.where(
                v00 == m, n00v,
                jnp.where(v01 == m, n00v + one,
                          jnp.where(v10 == m, n00v + hv, n00v + hv + one)))
            plsc.store_scatter(
                wtab, [iota, jnp.full((16,), m2, jnp.int32)], i2)
            rel = m2 - m1lo

            @pl.when((rel >= 0) & (rel < _STRIP))
            def _():
                murow[pl.ds(rel * _C, 16)] = m

            return 0

        lax.fori_loop(0, _HO, build_ox, 0)
        return 0

    lax.fori_loop(0, _HO, build_oy, 0)
    pltpu.sync_copy(murow,
                    muo_hbm.at[pl.ds(b * _OROW + m1lo * _C, _STRIP * _C)])

    def _fire(j, sem):
        nb = _nb(j)
        h = j % 2
        pltpu.async_copy(sig_hbm.at[b, pl.ds(nb, 2)],
                         quad.at[h, pl.ds(0, 2)], sem)
        pltpu.async_copy(sig_hbm.at[b, pl.ds(nb + _H, 2)],
                         quad.at[h, pl.ds(2, 2)], sem)

    def _drain(j, sem):
        h = j % 2
        for k in range(2):
            pltpu.make_async_copy(sig_hbm.at[b, pl.ds(0, 2)],
                                  quad.at[h, pl.ds(2 * k, 2)], sem).wait()

    def pair(j, _):
        even = (j % 2) == 0
        h = j % 2

        @pl.when(j + 1 < _STRIP)
        def _():
            @pl.when(even)
            def _():
                _fire(j + 1, semB)

            @pl.when(jnp.logical_not(even))
            def _():
                _fire(j + 1, semA)

        @pl.when(even)
        def _():
            _drain(j, semA)

        @pl.when(jnp.logical_not(even))
        def _():
            _drain(j, semB)

        m1 = m1lo + j
        nb = _nb(j)
        i1 = plsc.load_gather(wtab, [iota, jnp.full((16,), m1, jnp.int32)])
        r = i1 - jnp.full((16,), nb, jnp.int32)
        dy = r // jnp.full((16,), _H, jnp.int32)
        dx = r & jnp.full((16,), 1, jnp.int32)
        qselv = dy + dy + dx

        @pl.when((j >= 2) & even)
        def _():
            pltpu.make_async_copy(outrow.at[0], sigo_hbm.at[b, 0],
                                  semOA).wait()

        @pl.when((j >= 2) & jnp.logical_not(even))
        def _():
            pltpu.make_async_copy(outrow.at[1], sigo_hbm.at[b, 0],
                                  semOB).wait()

        hvv = jnp.full((16,), h, jnp.int32)
        for c in range(_C):
            qvv = jnp.full((16,), qselv[c], jnp.int32)
            cvv = jnp.full((16,), c, jnp.int32)

            for k in range(_NCH):
                off = min(k * 16, _M - 16)
                i2v = wtab[c, pl.ds(off, 16)]
                val = plsc.load_gather(quad, [hvv, qvv, cvv, i2v])
                outrow[h, c, pl.ds(off, 16)] = val

        @pl.when(even)
        def _():
            pltpu.async_copy(outrow.at[0], sigo_hbm.at[b, m1], semOA)

        @pl.when(jnp.logical_not(even))
        def _():
            pltpu.async_copy(outrow.at[1], sigo_hbm.at[b, m1], semOB)

        return 0

    lax.fori_loop(0, _STRIP, pair, 0)
    pltpu.make_async_copy(outrow.at[0], sigo_hbm.at[b, 0], semOA).wait()
    pltpu.make_async_copy(outrow.at[1], sigo_hbm.at[b, 0], semOB).wait()


@jax.jit
def _vdp_pool(mu_r, sig_t):
    mesh = plsc.VectorSubcoreMesh(core_axis_name="c", subcore_axis_name="s",
                                  num_cores=_NC, num_subcores=_NS)
    return pl.kernel(
        _sc_body,
        out_type=(
            jax.ShapeDtypeStruct((_B * _M * _C,), jnp.float32),
            jax.ShapeDtypeStruct((_B, _M, _C, _M), jnp.float32),
        ),
        mesh=mesh,
        compiler_params=pltpu.CompilerParams(needs_layout_passes=False),
        scratch_types=[
            pltpu.VMEM((2, 2 * _H * _C), jnp.float32),
            pltpu.VMEM((_C, _M), jnp.int32),
            pltpu.VMEM((_STRIP * _C,), jnp.float32),
            pltpu.VMEM((2, 4, _C, _N), jnp.float32),
            pltpu.VMEM((2, _C, _M), jnp.float32),
            pltpu.SemaphoreType.DMA,
            pltpu.SemaphoreType.DMA,
            pltpu.SemaphoreType.DMA,
            pltpu.SemaphoreType.DMA,
        ],
    )(mu_r, sig_t)


def kernel(mu_in, Sigma_in):
    mu_r = mu_in.reshape(_B * _H * _H * _C)
    sig_t = jnp.transpose(Sigma_in, (0, 1, 3, 2))
    muo, sigo = _vdp_pool(mu_r, sig_t)
    mu_out = muo.reshape(_B, _HO, _HO, _C)
    Sigma_out = jnp.transpose(sigo, (0, 1, 3, 2))
    return mu_out, Sigma_out
